# baseline (speedup 1.0000x reference)
import jax
import jax.numpy as jnp
from jax import lax
from jax.experimental import pallas as pl
from jax.experimental.pallas import tpu as pltpu
from jax.experimental.pallas import tpu_sc as plsc

_B = 16384
_F = 26
_V = 1000000
_D = 13
_S = 1 << 20


_NC = 2
_NS = 16
_NW = _NC * _NS
_RPW = _B // _NW
_IPW = _RPW * _F
_DPW = _RPW * _D
_JCH = _RPW // 16


_CRT = 32768


def _retile_body(tab_ref, out_ref):
    out_ref[...] = tab_ref[...].reshape(_F, _CRT // 128, 128)


def _retile(tables):
    return pl.pallas_call(
        _retile_body,
        grid=(_V // _CRT + 1,),
        in_specs=[pl.BlockSpec((_F, _CRT), lambda j: (0, j))],
        out_specs=pl.BlockSpec((_F, _CRT // 128, 128), lambda j: (0, j, 0)),
        out_shape=jax.ShapeDtypeStruct((_F, _S // 128, 128), jnp.float32),
    )(tables)


def _tec_body(idxT_hbm, tab_hbm, denT_hbm, dk_hbm, out_hbm,
              flat_v, vals_v, den_v, dk_v, out_v, gsem, dsem):
    c = lax.axis_index("c")
    s = lax.axis_index("s")
    wid = s * _NC + c
    base = wid * _RPW

    icps = [
        pltpu.async_copy(idxT_hbm.at[pl.ds(f * _B + base, _RPW)],
                         flat_v.at[pl.ds(f * _RPW, _RPW)], gsem)
        for f in range(_F)
    ]
    dcps = [
        pltpu.async_copy(denT_hbm.at[pl.ds(d * _B + base, _RPW)],
                         den_v.at[pl.ds(d * _RPW, _RPW)], dsem)
        for d in range(_D)
    ]
    dcps.append(pltpu.async_copy(dk_hbm, dk_v, dsem))
    for cp in icps:
        cp.wait()

    def _off_body(j, carry):
        o = j * 16
        for f in range(1, _F):
            plsc.addupdate(flat_v.at[pl.ds(f * _RPW + o, 16)],
                           jnp.full((16,), f * _S, jnp.int32))
        return carry

    lax.fori_loop(0, _JCH, _off_body, 0)

    pltpu.async_copy(tab_hbm.at[flat_v], vals_v, gsem).wait()
    for cp in dcps:
        cp.wait()
    dkv = dk_v[pl.ds(0, 16)]

    def _red_body(j, carry):
        o = j * 16
        acc = vals_v[pl.ds(o, 16)]
        for f in range(1, _F):
            acc = acc + vals_v[pl.ds(f * _RPW + o, 16)]
        for d in range(_D):
            acc = acc + den_v[pl.ds(d * _RPW + o, 16)] * dkv[d]
        out_v[pl.ds(o, 16)] = acc
        return carry

    lax.fori_loop(0, _JCH, _red_body, 0)

    pltpu.sync_copy(out_v, out_hbm.at[pl.ds(base, _RPW)])


@jax.jit
def kernel(sparse_idx, dense, tables, dense_kernel):
    idxT = sparse_idx.T.reshape(_F * _B)
    denT = dense.T.reshape(_D * _B)
    tab_flat = _retile(tables).reshape(_F * _S)
    dk16 = jnp.pad(dense_kernel.reshape(_D), (0, 16 - _D))

    mesh = plsc.VectorSubcoreMesh(core_axis_name="c", subcore_axis_name="s")
    run = pl.kernel(
        _tec_body,
        out_type=jax.ShapeDtypeStruct((_B,), jnp.float32),
        mesh=mesh,
        scratch_types=[
            pltpu.VMEM((_IPW,), jnp.int32),
            pltpu.VMEM((_IPW,), jnp.float32),
            pltpu.VMEM((_DPW,), jnp.float32),
            pltpu.VMEM((16,), jnp.float32),
            pltpu.VMEM((_RPW,), jnp.float32),
            pltpu.SemaphoreType.DMA,
            pltpu.SemaphoreType.DMA,
        ],
    )
    out = run(idxT, tab_flat, denT, dk16)
    return out.reshape(_B, 1)


# speedup vs baseline: 1.0052x; 1.0052x over previous
import jax
import jax.numpy as jnp
from jax import lax
from jax.experimental import pallas as pl
from jax.experimental.pallas import tpu as pltpu
from jax.experimental.pallas import tpu_sc as plsc

_B = 16384
_F = 26
_V = 1000000
_D = 13
_S = 1 << 20


_NC = 2
_NS = 16
_NW = _NC * _NS
_RPW = _B // _NW
_IPW = _RPW * _F
_DPW = _RPW * _D
_JCH = _RPW // 16


_CRT = 65536


def _retile_body(tab_ref, out_ref):
    out_ref[...] = tab_ref[...].reshape(_F, _CRT // 128, 128)


def _retile(tables):
    return pl.pallas_call(
        _retile_body,
        grid=(_V // _CRT + 1,),
        in_specs=[pl.BlockSpec((_F, _CRT), lambda j: (0, j))],
        out_specs=pl.BlockSpec((_F, _CRT // 128, 128), lambda j: (0, j, 0)),
        out_shape=jax.ShapeDtypeStruct((_F, _S // 128, 128), jnp.float32),
    )(tables)


def _tec_body(idxT_hbm, tab_hbm, denT_hbm, dk_hbm, out_hbm,
              flat_v, vals_v, den_v, dk_v, out_v, gsem, dsem):
    c = lax.axis_index("c")
    s = lax.axis_index("s")
    wid = s * _NC + c
    base = wid * _RPW

    icps = [
        pltpu.async_copy(idxT_hbm.at[pl.ds(f * _B + base, _RPW)],
                         flat_v.at[pl.ds(f * _RPW, _RPW)], gsem)
        for f in range(_F)
    ]
    dcps = [
        pltpu.async_copy(denT_hbm.at[pl.ds(d * _B + base, _RPW)],
                         den_v.at[pl.ds(d * _RPW, _RPW)], dsem)
        for d in range(_D)
    ]
    dcps.append(pltpu.async_copy(dk_hbm, dk_v, dsem))

    gcps = []
    for f in range(_F):
        icps[f].wait()
        gcps.append(pltpu.async_copy(
            tab_hbm.at[pl.ds(f * _S, _S)].at[flat_v.at[pl.ds(f * _RPW, _RPW)]],
            vals_v.at[pl.ds(f * _RPW, _RPW)], gsem))
    for cp in gcps:
        cp.wait()
    for cp in dcps:
        cp.wait()
    dkv = dk_v[pl.ds(0, 16)]

    def _red_body(j, carry):
        o = j * 16
        acc = vals_v[pl.ds(o, 16)]
        for f in range(1, _F):
            acc = acc + vals_v[pl.ds(f * _RPW + o, 16)]
        for d in range(_D):
            acc = acc + den_v[pl.ds(d * _RPW + o, 16)] * dkv[d]
        out_v[pl.ds(o, 16)] = acc
        return carry

    lax.fori_loop(0, _JCH, _red_body, 0)

    pltpu.sync_copy(out_v, out_hbm.at[pl.ds(base, _RPW)])


@jax.jit
def kernel(sparse_idx, dense, tables, dense_kernel):
    idxT = sparse_idx.T.reshape(_F * _B)
    denT = dense.T.reshape(_D * _B)
    tab_flat = _retile(tables).reshape(_F * _S)
    dk16 = jnp.pad(dense_kernel.reshape(_D), (0, 16 - _D))

    mesh = plsc.VectorSubcoreMesh(core_axis_name="c", subcore_axis_name="s")
    run = pl.kernel(
        _tec_body,
        out_type=jax.ShapeDtypeStruct((_B,), jnp.float32),
        mesh=mesh,
        scratch_types=[
            pltpu.VMEM((_IPW,), jnp.int32),
            pltpu.VMEM((_IPW,), jnp.float32),
            pltpu.VMEM((_DPW,), jnp.float32),
            pltpu.VMEM((16,), jnp.float32),
            pltpu.VMEM((_RPW,), jnp.float32),
            pltpu.SemaphoreType.DMA,
            pltpu.SemaphoreType.DMA,
        ],
    )
    out = run(idxT, tab_flat, denT, dk16)
    return out.reshape(_B, 1)


# speedup vs baseline: 1.0130x; 1.0077x over previous
import jax
import jax.numpy as jnp
from jax import lax
from jax.experimental import pallas as pl
from jax.experimental.pallas import tpu as pltpu
from jax.experimental.pallas import tpu_sc as plsc

_B = 16384
_F = 26
_V = 1000000
_D = 13
_S = 1 << 20


_NC = 2
_NS = 16
_NW = _NC * _NS
_RPW = _B // _NW
_IPW = _RPW * _F
_DPW = _RPW * _D
_JCH = _RPW // 16


_CRT = 65536


def _retile_body(tab_ref, out_ref):
    out_ref[...] = tab_ref[...].reshape(_F, _CRT // 128, 128)


def _retile(tables):
    return pl.pallas_call(
        _retile_body,
        grid=(_V // _CRT + 1,),
        in_specs=[pl.BlockSpec((_F, _CRT), lambda j: (0, j))],
        out_specs=pl.BlockSpec((_F, _CRT // 128, 128), lambda j: (0, j, 0)),
        out_shape=jax.ShapeDtypeStruct((_F, _S // 128, 128), jnp.float32),
    )(tables)


def _tec_body(idxT_hbm, tab_hbm, denT_hbm, dk_hbm, out_hbm,
              flat_v, vals_v, den_v, dk_v, out_v, gsem, dsem):
    c = lax.axis_index("c")
    s = lax.axis_index("s")
    wid = s * _NC + c
    base = wid * _RPW

    icps = [
        pltpu.async_copy(idxT_hbm.at[pl.ds(f * _B + base, _RPW)],
                         flat_v.at[pl.ds(f * _RPW, _RPW)], gsem)
        for f in range(_F)
    ]
    dcps = [
        pltpu.async_copy(denT_hbm.at[pl.ds(d * _B + base, _RPW)],
                         den_v.at[pl.ds(d * _RPW, _RPW)], dsem)
        for d in range(_D)
    ]
    dcps.append(pltpu.async_copy(dk_hbm, dk_v, dsem))

    for cp in icps:
        cp.wait()

    def _off_body(j, carry):
        o = j * 16
        for f in range(1, _F):
            plsc.addupdate(flat_v.at[pl.ds(f * _RPW + o, 16)],
                           jnp.full((16,), f * _S, jnp.int32))
        return carry

    lax.fori_loop(0, _JCH, _off_body, 0)

    pltpu.async_copy(tab_hbm.at[flat_v], vals_v, gsem).wait()
    for cp in dcps:
        cp.wait()
    dkv = dk_v[pl.ds(0, 16)]

    def _red_body(j, carry):
        o = j * 16
        acc = vals_v[pl.ds(o, 16)]
        for f in range(1, _F):
            acc = acc + vals_v[pl.ds(f * _RPW + o, 16)]
        for d in range(_D):
            acc = acc + den_v[pl.ds(d * _RPW + o, 16)] * dkv[d]
        out_v[pl.ds(o, 16)] = acc
        return carry

    lax.fori_loop(0, _JCH, _red_body, 0)

    pltpu.sync_copy(out_v, out_hbm.at[pl.ds(base, _RPW)])


@jax.jit
def kernel(sparse_idx, dense, tables, dense_kernel):
    idxT = sparse_idx.T.reshape(_F * _B)
    denT = dense.T.reshape(_D * _B)
    tab_flat = _retile(tables).reshape(_F * _S)
    dk16 = jnp.pad(dense_kernel.reshape(_D), (0, 16 - _D))

    mesh = plsc.VectorSubcoreMesh(core_axis_name="c", subcore_axis_name="s")
    run = pl.kernel(
        _tec_body,
        out_type=jax.ShapeDtypeStruct((_B,), jnp.float32),
        mesh=mesh,
        scratch_types=[
            pltpu.VMEM((_IPW,), jnp.int32),
            pltpu.VMEM((_IPW,), jnp.float32),
            pltpu.VMEM((_DPW,), jnp.float32),
            pltpu.VMEM((16,), jnp.float32),
            pltpu.VMEM((_RPW,), jnp.float32),
            pltpu.SemaphoreType.DMA,
            pltpu.SemaphoreType.DMA,
        ],
    )
    out = run(idxT, tab_flat, denT, dk16)
    return out.reshape(_B, 1)
